# Initial kernel scaffold; baseline (speedup 1.0000x reference)
import jax
import jax.numpy as jnp
from jax.experimental import pallas as pl


def kernel(edge_index, edge_type, basis1, comp1, root1, bias1, basis2, comp2, root2, bias2, basis3, comp3, root3, bias3):
    raise NotImplementedError("write your pallas kernel here")


# speedup vs baseline: 81.1445x; 81.1445x over previous
import functools

import jax
import jax.numpy as jnp
from jax import lax
from jax.experimental import pallas as pl
from jax.experimental.pallas import tpu as pltpu
from jax.experimental.pallas import tpu_sc as plsc

_N = 10000
_R = 70
_B = 30
_E = 320000

_NC = 2
_NS = 16
_NW = _NC * _NS
_CH = 80
_EPT = _E // _NS
_EPW = _E // _NW
_CNT = _N * _R
_CNT_PAD = 700032
_ZCH = _CNT_PAD // _NS

_mesh = plsc.VectorSubcoreMesh(
    core_axis_name="c", subcore_axis_name="s", num_cores=_NC, num_subcores=_NS)
_sc_params = pltpu.CompilerParams(needs_layout_passes=False,
                                  use_tc_tiling_on_sc=False)


def _idx_body(s_ref, d_ref, t_ref, i1_ref, ic_ref):
    i1_ref[...] = t_ref[...] * _N + s_ref[...]
    ic_ref[...] = d_ref[...] * _R + t_ref[...]


_tc_idx = pl.pallas_call(
    _idx_body,
    out_shape=(jax.ShapeDtypeStruct((2500, 128), jnp.int32),
               jax.ShapeDtypeStruct((2500, 128), jnp.int32)))


def _t1_body(c_ref, b_ref, o_ref):
    o_ref[...] = jnp.dot(c_ref[...], b_ref[...],
                         preferred_element_type=jnp.float32,
                         precision=lax.Precision.HIGHEST)


_tc_t1 = pl.pallas_call(
    _t1_body,
    grid=(125,),
    in_specs=[pl.BlockSpec((_R, _B), lambda i: (0, 0)),
              pl.BlockSpec((_B, 2560), lambda i: (0, i))],
    out_specs=pl.BlockSpec((_R, 2560), lambda i: (0, i)),
    out_shape=jax.ShapeDtypeStruct((_R, _E), jnp.float32))


def _c1_body(p_ref, r1_ref, b1_ref, rt2_ref, x_ref, r2_ref):
    x = jnp.maximum(p_ref[0] + p_ref[1] + r1_ref[...] + b1_ref[...], 0.0)
    x_ref[...] = x
    r2_ref[...] = jnp.dot(x, rt2_ref[...], preferred_element_type=jnp.float32,
                         precision=lax.Precision.HIGHEST)


_tc_c1 = pl.pallas_call(
    _c1_body,
    out_shape=(jax.ShapeDtypeStruct((_N, 32), jnp.float32),
               jax.ShapeDtypeStruct((_N, 8), jnp.float32)))


def _t2_body(x_ref, c2_ref, b2_ref, o_ref):
    r = pl.program_id(0)
    rmask = lax.broadcasted_iota(jnp.int32, (_R, _B), 0) == r
    crow = jnp.sum(jnp.where(rmask, c2_ref[...], 0.0), axis=0)
    w2 = jnp.sum(crow[:, None, None] * b2_ref[...], axis=0)
    xs = x_ref[...]
    o_ref[...] = jnp.concatenate(
        [jnp.dot(xs[:, k * 32:(k + 1) * 32], w2,
                 preferred_element_type=jnp.float32,
                         precision=lax.Precision.HIGHEST) for k in range(16)],
        axis=1)[None]


_tc_t2 = pl.pallas_call(
    _t2_body,
    grid=(_R,),
    in_specs=[pl.BlockSpec((625, 512), lambda r: (0, 0)),
              pl.BlockSpec((_R, _B), lambda r: (0, 0)),
              pl.BlockSpec((_B, 32, 8), lambda r: (0, 0, 0))],
    out_specs=pl.BlockSpec((1, 625, 128), lambda r: (r, 0, 0)),
    out_shape=jax.ShapeDtypeStruct((_R, 625, 128), jnp.float32))


def _c2_body(p_ref, r2_ref, b2_ref, rt3_ref, x_ref, r3_ref):
    x = jnp.maximum(p_ref[0] + p_ref[1] + r2_ref[...] + b2_ref[...], 0.0)
    x_ref[...] = x
    r3_ref[...] = jnp.dot(x, rt3_ref[...], preferred_element_type=jnp.float32,
                         precision=lax.Precision.HIGHEST)


_tc_c2 = pl.pallas_call(
    _c2_body,
    out_shape=(jax.ShapeDtypeStruct((_N, 8), jnp.float32),
               jax.ShapeDtypeStruct((_N, 1), jnp.float32)))


def _t3_body(c3_ref, b3_ref, x_ref, o_ref):
    w3 = jnp.dot(c3_ref[...], b3_ref[...], preferred_element_type=jnp.float32,
                         precision=lax.Precision.HIGHEST)
    o_ref[...] = lax.dot_general(w3, x_ref[...], (((1,), (1,)), ((), ())),
                                 preferred_element_type=jnp.float32,
                         precision=lax.Precision.HIGHEST)


_tc_t3 = pl.pallas_call(
    _t3_body,
    out_shape=jax.ShapeDtypeStruct((_R, _N), jnp.float32))


def _fin_body(p_ref, r3_ref, b3_ref, o_ref):
    o_ref[...] = p_ref[0] + p_ref[1] + r3_ref[...] + b3_ref[...]


_tc_fin = pl.pallas_call(
    _fin_body,
    out_shape=jax.ShapeDtypeStruct((_N, 1), jnp.float32))


def _hist_body(idxc_hbm, zeros_hbm, scale_hbm,
               icbuf, ones_v, cvals, sbuf, zbuf, cnt_sp, sem):
    c = lax.axis_index("c")
    s = lax.axis_index("s")
    pltpu.sync_copy(zeros_hbm, zbuf)
    pltpu.sync_copy(zbuf, cnt_sp.at[pl.ds(s * _ZCH, _ZCH)])
    for g in range(_CH // 16):
        ones_v[pl.ds(g * 16, 16)] = jnp.ones((16,), jnp.float32)
    plsc.subcore_barrier()

    def hist_chunk(k, carry):
        b = s * _EPT + k * _CH
        pltpu.sync_copy(idxc_hbm.at[pl.ds(b, _CH)], icbuf)
        pltpu.sync_copy(ones_v, cnt_sp.at[icbuf], add=True)
        return carry

    lax.fori_loop(0, _EPT // _CH, hist_chunk, 0)
    plsc.subcore_barrier()

    w = c * _NS + s

    def scale_chunk(k, carry):
        b = w * _EPW + k * _CH
        pltpu.sync_copy(idxc_hbm.at[pl.ds(b, _CH)], icbuf)
        pltpu.async_copy(cnt_sp.at[icbuf], cvals, sem).wait()
        for g in range(_CH // 16):
            v = cvals[pl.ds(g * 16, 16)]
            sbuf[pl.ds(g * 16, 16)] = 1.0 / jnp.maximum(v, 1.0)
        pltpu.sync_copy(sbuf, scale_hbm.at[pl.ds(b, _CH)])
        return carry

    lax.fori_loop(0, _EPW // _CH, scale_chunk, 0)


_sc_hist = pl.kernel(
    _hist_body,
    out_type=jax.ShapeDtypeStruct((_E,), jnp.float32),
    mesh=_mesh,
    compiler_params=_sc_params,
    scratch_types=[
        pltpu.VMEM((_CH,), jnp.int32),
        pltpu.VMEM((_CH,), jnp.float32),
        pltpu.VMEM((_CH,), jnp.float32),
        pltpu.VMEM((_CH,), jnp.float32),
        pltpu.VMEM((_ZCH,), jnp.float32),
        pltpu.VMEM_SHARED((_CNT_PAD,), jnp.float32),
        pltpu.SemaphoreType.DMA,
    ])


def _make_sc_layer(C):
    def body(t_hbm, idx1_hbm, dst_hbm, scale_hbm, zeros_hbm,
             out_hbm, igbuf, dbuf, sbuf, rows, zobuf, acc_sp, sem):
        c = lax.axis_index("c")
        s = lax.axis_index("s")
        w = c * _NS + s

        pltpu.sync_copy(zeros_hbm, zobuf)

        @pl.when(s < 15)
        def _():
            pltpu.sync_copy(zobuf.at[pl.ds(0, 624)],
                            acc_sp.at[pl.ds(s * 624, 624)])

        @pl.when(s == 15)
        def _():
            pltpu.sync_copy(zobuf, acc_sp.at[pl.ds(9360, 640)])

        plsc.subcore_barrier()

        iota = lax.iota(jnp.int32, 16)

        def chunk(k, carry):
            b = w * _EPW + k * _CH
            pltpu.sync_copy(idx1_hbm.at[pl.ds(b, _CH)], igbuf)
            pltpu.sync_copy(dst_hbm.at[pl.ds(b, _CH)], dbuf)
            pltpu.sync_copy(scale_hbm.at[pl.ds(b, _CH)], sbuf)
            pltpu.async_copy(t_hbm.at[igbuf], rows, sem).wait()
            for p in range(_CH // 2):
                e = 2 * p + (iota >> 3)
                sc = plsc.load_gather(sbuf, [e])
                for cb in range(C // 8):
                    cv = cb * 8 + (iota & 7)
                    val = plsc.load_gather(rows, [e, cv])
                    plsc.store_scatter(rows, [e, cv], val * sc)
            pltpu.sync_copy(rows, acc_sp.at[dbuf], add=True)
            return carry

        lax.fori_loop(0, _EPW // _CH, chunk, 0)
        plsc.subcore_barrier()

        @pl.when(s < 15)
        def _():
            pltpu.sync_copy(acc_sp.at[pl.ds(s * 624, 624)],
                            zobuf.at[pl.ds(0, 624)])
            pltpu.sync_copy(zobuf.at[pl.ds(0, 624)],
                            out_hbm.at[c, pl.ds(s * 624, 624)])

        @pl.when(s == 15)
        def _():
            pltpu.sync_copy(acc_sp.at[pl.ds(9360, 640)], zobuf)
            pltpu.sync_copy(zobuf, out_hbm.at[c, pl.ds(9360, 640)])

    return pl.kernel(
        body,
        out_type=jax.ShapeDtypeStruct((_NC, _N, C), jnp.float32),
        mesh=_mesh,
        compiler_params=_sc_params,
        scratch_types=[
            pltpu.VMEM((_CH,), jnp.int32),
            pltpu.VMEM((_CH,), jnp.int32),
            pltpu.VMEM((_CH,), jnp.float32),
            pltpu.VMEM((_CH, C), jnp.float32),
            pltpu.VMEM((640, C), jnp.float32),
            pltpu.VMEM_SHARED((_N, C), jnp.float32),
            pltpu.SemaphoreType.DMA,
        ])


_sc_layer32 = _make_sc_layer(32)
_sc_layer8 = _make_sc_layer(8)


def _l3_body(t_hbm, idx1_hbm, dst_hbm, scale_hbm, zeros_hbm,
             out_hbm, igbuf, dbuf, sbuf, rows, zobuf, acc_sp, sem):
    c = lax.axis_index("c")
    s = lax.axis_index("s")
    w = c * _NS + s
    @pl.when(s < 5)
    def _():
        pltpu.sync_copy(zeros_hbm, zobuf)
        pltpu.sync_copy(zobuf, acc_sp.at[pl.ds(s * 2000, 2000)])
    plsc.subcore_barrier()

    def chunk(k, carry):
        b = w * _EPW + k * _CH
        pltpu.sync_copy(idx1_hbm.at[pl.ds(b, _CH)], igbuf)
        pltpu.sync_copy(dst_hbm.at[pl.ds(b, _CH)], dbuf)
        pltpu.sync_copy(scale_hbm.at[pl.ds(b, _CH)], sbuf)
        pltpu.async_copy(t_hbm.at[igbuf], rows, sem).wait()
        for g in range(_CH // 16):
            sl = pl.ds(g * 16, 16)
            rows[sl] = rows[sl] * sbuf[sl]
        pltpu.sync_copy(rows, acc_sp.at[dbuf], add=True)
        return carry

    lax.fori_loop(0, _EPW // _CH, chunk, 0)
    plsc.subcore_barrier()

    @pl.when(s < 5)
    def _():
        pltpu.sync_copy(acc_sp.at[pl.ds(s * 2000, 2000)], zobuf)
        pltpu.sync_copy(zobuf, out_hbm.at[pl.ds(c * _N + s * 2000, 2000)])


_sc_layer1 = pl.kernel(
    _l3_body,
    out_type=jax.ShapeDtypeStruct((_NC * _N,), jnp.float32),
    mesh=_mesh,
    compiler_params=_sc_params,
    scratch_types=[
        pltpu.VMEM((_CH,), jnp.int32),
        pltpu.VMEM((_CH,), jnp.int32),
        pltpu.VMEM((_CH,), jnp.float32),
        pltpu.VMEM((_CH,), jnp.float32),
        pltpu.VMEM((2000,), jnp.float32),
        pltpu.VMEM_SHARED((_N,), jnp.float32),
        pltpu.SemaphoreType.DMA,
    ])


def kernel(edge_index, edge_type, basis1, comp1, root1, bias1,
           basis2, comp2, root2, bias2, basis3, comp3, root3, bias3):
    src = edge_index[0]
    dst = edge_index[1]

    idx1_2d, idxc_2d = _tc_idx(src.reshape(2500, 128),
                               dst.reshape(2500, 128),
                               edge_type.reshape(2500, 128))
    idx1 = idx1_2d.reshape(_E)
    idxc = idxc_2d.reshape(_E)

    scale = _sc_hist(idxc, jnp.zeros((_ZCH,), jnp.float32))

    t1 = _tc_t1(comp1, basis1.reshape(_B, _E))
    p1 = _sc_layer32(t1.reshape(_R * _N, 32), idx1, dst, scale,
                     jnp.zeros((640, 32), jnp.float32))
    x1, r2term = _tc_c1(p1, root1, bias1.reshape(1, 32), root2)

    t2 = _tc_t2(x1.reshape(625, 512), comp2, basis2)
    p2 = _sc_layer8(t2.reshape(_R * _N, 8), idx1, dst, scale,
                    jnp.zeros((640, 8), jnp.float32))
    x2, r3term = _tc_c2(p2, r2term, bias2.reshape(1, 8), root3)

    t3 = _tc_t3(comp3, basis3.reshape(_B, 8), x2)
    p3 = _sc_layer1(t3.reshape(_R * _N), idx1, dst, scale,
                    jnp.zeros((2000,), jnp.float32))
    return _tc_fin(p3.reshape(_NC, _N, 1), r3term, bias3.reshape(1, 1))


# speedup vs baseline: 105.2251x; 1.2968x over previous
import functools

import jax
import jax.numpy as jnp
from jax import lax
from jax.experimental import pallas as pl
from jax.experimental.pallas import tpu as pltpu
from jax.experimental.pallas import tpu_sc as plsc

_N = 10000
_R = 70
_B = 30
_E = 320000

_NC = 2
_NS = 16
_NW = _NC * _NS
_CH = 80
_EPT = _E // _NS
_EPW = _E // _NW
_CNT = _N * _R
_CNT_PAD = 700032
_ZCH = _CNT_PAD // _NS

_mesh = plsc.VectorSubcoreMesh(
    core_axis_name="c", subcore_axis_name="s", num_cores=_NC, num_subcores=_NS)
_sc_params = pltpu.CompilerParams(needs_layout_passes=False,
                                  use_tc_tiling_on_sc=False)


def _idx_body(s_ref, d_ref, t_ref, i1_ref, ic_ref):
    i1_ref[...] = t_ref[...] * _N + s_ref[...]
    ic_ref[...] = d_ref[...] * _R + t_ref[...]


_tc_idx = pl.pallas_call(
    _idx_body,
    out_shape=(jax.ShapeDtypeStruct((2500, 128), jnp.int32),
               jax.ShapeDtypeStruct((2500, 128), jnp.int32)))


def _t1_body(c_ref, b_ref, o_ref):
    o_ref[...] = jnp.dot(c_ref[...], b_ref[...],
                         preferred_element_type=jnp.float32,
                         precision=lax.Precision.HIGHEST)


_tc_t1 = pl.pallas_call(
    _t1_body,
    grid=(125,),
    in_specs=[pl.BlockSpec((_R, _B), lambda i: (0, 0)),
              pl.BlockSpec((_B, 2560), lambda i: (0, i))],
    out_specs=pl.BlockSpec((_R, 2560), lambda i: (0, i)),
    out_shape=jax.ShapeDtypeStruct((_R, _E), jnp.float32))


def _c1_body(p_ref, r1_ref, b1_ref, rt2_ref, x_ref, r2_ref):
    x = jnp.maximum(p_ref[0] + p_ref[1] + r1_ref[...] + b1_ref[...], 0.0)
    x_ref[...] = x
    r2_ref[...] = jnp.dot(x, rt2_ref[...], preferred_element_type=jnp.float32,
                         precision=lax.Precision.HIGHEST)


_tc_c1 = pl.pallas_call(
    _c1_body,
    out_shape=(jax.ShapeDtypeStruct((_N, 32), jnp.float32),
               jax.ShapeDtypeStruct((_N, 8), jnp.float32)))


def _t2_body(x_ref, c2_ref, b2_ref, o_ref):
    r = pl.program_id(0)
    rmask = lax.broadcasted_iota(jnp.int32, (_R, _B), 0) == r
    crow = jnp.sum(jnp.where(rmask, c2_ref[...], 0.0), axis=0)
    w2 = jnp.sum(crow[:, None, None] * b2_ref[...], axis=0)
    xs = x_ref[...]
    o_ref[...] = jnp.concatenate(
        [jnp.dot(xs[:, k * 32:(k + 1) * 32], w2,
                 preferred_element_type=jnp.float32,
                         precision=lax.Precision.HIGHEST) for k in range(16)],
        axis=1)[None]


_tc_t2 = pl.pallas_call(
    _t2_body,
    grid=(_R,),
    in_specs=[pl.BlockSpec((625, 512), lambda r: (0, 0)),
              pl.BlockSpec((_R, _B), lambda r: (0, 0)),
              pl.BlockSpec((_B, 32, 8), lambda r: (0, 0, 0))],
    out_specs=pl.BlockSpec((1, 625, 128), lambda r: (r, 0, 0)),
    out_shape=jax.ShapeDtypeStruct((_R, 625, 128), jnp.float32))


def _c2_body(p_ref, r2_ref, b2_ref, rt3_ref, x_ref, r3_ref):
    x = jnp.maximum(p_ref[0] + p_ref[1] + r2_ref[...] + b2_ref[...], 0.0)
    x_ref[...] = x
    r3_ref[...] = jnp.dot(x, rt3_ref[...], preferred_element_type=jnp.float32,
                         precision=lax.Precision.HIGHEST)


_tc_c2 = pl.pallas_call(
    _c2_body,
    out_shape=(jax.ShapeDtypeStruct((_N, 8), jnp.float32),
               jax.ShapeDtypeStruct((_N, 1), jnp.float32)))


def _t3_body(c3_ref, b3_ref, x_ref, o_ref):
    w3 = jnp.dot(c3_ref[...], b3_ref[...], preferred_element_type=jnp.float32,
                         precision=lax.Precision.HIGHEST)
    o_ref[...] = lax.dot_general(w3, x_ref[...], (((1,), (1,)), ((), ())),
                                 preferred_element_type=jnp.float32,
                         precision=lax.Precision.HIGHEST)


_tc_t3 = pl.pallas_call(
    _t3_body,
    out_shape=jax.ShapeDtypeStruct((_R, _N), jnp.float32))


def _fin_body(p_ref, r3_ref, b3_ref, o_ref):
    o_ref[...] = p_ref[0] + p_ref[1] + r3_ref[...] + b3_ref[...]


_tc_fin = pl.pallas_call(
    _fin_body,
    out_shape=jax.ShapeDtypeStruct((_N, 1), jnp.float32))


def _hist_body(idxc2_hbm, zeros_hbm, scale_hbm,
               ic2d, ic2g, ones_v, cvals, sall, zbuf, cnt_sp, sem):
    c = lax.axis_index("c")
    s = lax.axis_index("s")
    pltpu.sync_copy(zeros_hbm, zbuf)
    pltpu.sync_copy(zbuf, cnt_sp.at[pl.ds(s * _ZCH, _ZCH)])
    for g in range(_CH // 16):
        ones_v[pl.ds(g * 16, 16)] = jnp.ones((16,), jnp.float32)
    nch = _EPT // _CH
    ncw = _EPW // _CH
    w = c * _NS + s
    pltpu.sync_copy(idxc2_hbm.at[pl.ds(s * nch, nch)], ic2d)
    pltpu.sync_copy(idxc2_hbm.at[pl.ds(w * ncw, ncw)], ic2g)
    plsc.subcore_barrier()

    def hist_chunk(k, carry):
        pltpu.sync_copy(ones_v, cnt_sp.at[ic2d.at[k]], add=True)
        return carry

    lax.fori_loop(0, nch, hist_chunk, 0)
    plsc.subcore_barrier()

    def scale_chunk(k, carry):
        pltpu.async_copy(cnt_sp.at[ic2g.at[k]], cvals, sem).wait()
        for g in range(_CH // 16):
            v = cvals[pl.ds(g * 16, 16)]
            sall[pl.ds(k * _CH + g * 16, 16)] = 1.0 / jnp.maximum(v, 1.0)
        return carry

    lax.fori_loop(0, ncw, scale_chunk, 0)
    pltpu.sync_copy(sall, scale_hbm.at[pl.ds(w * _EPW, _EPW)])


_sc_hist = pl.kernel(
    _hist_body,
    out_type=jax.ShapeDtypeStruct((_E,), jnp.float32),
    mesh=_mesh,
    compiler_params=_sc_params,
    scratch_types=[
        pltpu.VMEM((_EPT // _CH, _CH), jnp.int32),
        pltpu.VMEM((_EPW // _CH, _CH), jnp.int32),
        pltpu.VMEM((_CH,), jnp.float32),
        pltpu.VMEM((_CH,), jnp.float32),
        pltpu.VMEM((_EPW,), jnp.float32),
        pltpu.VMEM((_ZCH,), jnp.float32),
        pltpu.VMEM_SHARED((_CNT_PAD,), jnp.float32),
        pltpu.SemaphoreType.DMA,
    ])


def _make_sc_layer(C):
    def body(t_hbm, idx12_hbm, dst2_hbm, scale_hbm, zeros_hbm,
             out_hbm, ig2d, d2d, sall, rows, zobuf, acc_sp, sem):
        c = lax.axis_index("c")
        s = lax.axis_index("s")
        w = c * _NS + s
        ncw = _EPW // _CH

        pltpu.sync_copy(zeros_hbm, zobuf)
        pltpu.sync_copy(idx12_hbm.at[pl.ds(w * ncw, ncw)], ig2d)
        pltpu.sync_copy(dst2_hbm.at[pl.ds(w * ncw, ncw)], d2d)
        pltpu.sync_copy(scale_hbm.at[pl.ds(w * _EPW, _EPW)], sall)

        @pl.when(s < 15)
        def _():
            pltpu.sync_copy(zobuf.at[pl.ds(0, 624)],
                            acc_sp.at[pl.ds(s * 624, 624)])

        @pl.when(s == 15)
        def _():
            pltpu.sync_copy(zobuf, acc_sp.at[pl.ds(9360, 640)])

        plsc.subcore_barrier()

        iota = lax.iota(jnp.int32, 16)

        def chunk(k, carry):
            pltpu.async_copy(t_hbm.at[ig2d.at[k]], rows, sem).wait()
            for p in range(_CH // 2):
                e = 2 * p + (iota >> 3)
                sc = plsc.load_gather(sall, [k * _CH + e])
                for cb in range(C // 8):
                    cv = cb * 8 + (iota & 7)
                    val = plsc.load_gather(rows, [e, cv])
                    plsc.store_scatter(rows, [e, cv], val * sc)
            pltpu.sync_copy(rows, acc_sp.at[d2d.at[k]], add=True)
            return carry

        lax.fori_loop(0, ncw, chunk, 0)
        plsc.subcore_barrier()

        @pl.when(s < 15)
        def _():
            pltpu.sync_copy(acc_sp.at[pl.ds(s * 624, 624)],
                            zobuf.at[pl.ds(0, 624)])
            pltpu.sync_copy(zobuf.at[pl.ds(0, 624)],
                            out_hbm.at[c, pl.ds(s * 624, 624)])

        @pl.when(s == 15)
        def _():
            pltpu.sync_copy(acc_sp.at[pl.ds(9360, 640)], zobuf)
            pltpu.sync_copy(zobuf, out_hbm.at[c, pl.ds(9360, 640)])

    return pl.kernel(
        body,
        out_type=jax.ShapeDtypeStruct((_NC, _N, C), jnp.float32),
        mesh=_mesh,
        compiler_params=_sc_params,
        scratch_types=[
            pltpu.VMEM((_EPW // _CH, _CH), jnp.int32),
            pltpu.VMEM((_EPW // _CH, _CH), jnp.int32),
            pltpu.VMEM((_EPW,), jnp.float32),
            pltpu.VMEM((_CH, C), jnp.float32),
            pltpu.VMEM((640, C), jnp.float32),
            pltpu.VMEM_SHARED((_N, C), jnp.float32),
            pltpu.SemaphoreType.DMA,
        ])


_sc_layer32 = _make_sc_layer(32)
_sc_layer8 = _make_sc_layer(8)


def _l3_body(t_hbm, idx12_hbm, dst2_hbm, scale_hbm, zeros_hbm,
             out_hbm, ig2d, d2d, sall, rows, zobuf, acc_sp, sem):
    c = lax.axis_index("c")
    s = lax.axis_index("s")
    w = c * _NS + s
    ncw = _EPW // _CH
    pltpu.sync_copy(idx12_hbm.at[pl.ds(w * ncw, ncw)], ig2d)
    pltpu.sync_copy(dst2_hbm.at[pl.ds(w * ncw, ncw)], d2d)
    pltpu.sync_copy(scale_hbm.at[pl.ds(w * _EPW, _EPW)], sall)
    @pl.when(s < 5)
    def _():
        pltpu.sync_copy(zeros_hbm, zobuf)
        pltpu.sync_copy(zobuf, acc_sp.at[pl.ds(s * 2000, 2000)])
    plsc.subcore_barrier()

    def chunk(k, carry):
        pltpu.async_copy(t_hbm.at[ig2d.at[k]], rows, sem).wait()
        for g in range(_CH // 16):
            sl = pl.ds(g * 16, 16)
            ssl = pl.ds(k * _CH + g * 16, 16)
            rows[sl] = rows[sl] * sall[ssl]
        pltpu.sync_copy(rows, acc_sp.at[d2d.at[k]], add=True)
        return carry

    lax.fori_loop(0, ncw, chunk, 0)
    plsc.subcore_barrier()

    @pl.when(s < 5)
    def _():
        pltpu.sync_copy(acc_sp.at[pl.ds(s * 2000, 2000)], zobuf)
        pltpu.sync_copy(zobuf, out_hbm.at[pl.ds(c * _N + s * 2000, 2000)])


_sc_layer1 = pl.kernel(
    _l3_body,
    out_type=jax.ShapeDtypeStruct((_NC * _N,), jnp.float32),
    mesh=_mesh,
    compiler_params=_sc_params,
    scratch_types=[
        pltpu.VMEM((_EPW // _CH, _CH), jnp.int32),
        pltpu.VMEM((_EPW // _CH, _CH), jnp.int32),
        pltpu.VMEM((_EPW,), jnp.float32),
        pltpu.VMEM((_CH,), jnp.float32),
        pltpu.VMEM((2000,), jnp.float32),
        pltpu.VMEM_SHARED((_N,), jnp.float32),
        pltpu.SemaphoreType.DMA,
    ])


def kernel(edge_index, edge_type, basis1, comp1, root1, bias1,
           basis2, comp2, root2, bias2, basis3, comp3, root3, bias3):
    src = edge_index[0]
    dst = edge_index[1]

    idx1_2d, idxc_2d = _tc_idx(src.reshape(2500, 128),
                               dst.reshape(2500, 128),
                               edge_type.reshape(2500, 128))
    idx1c = idx1_2d.reshape(_E // _CH, _CH)
    idxcc = idxc_2d.reshape(_E // _CH, _CH)
    dstc = dst.reshape(_E // _CH, _CH)

    scale = _sc_hist(idxcc, jnp.zeros((_ZCH,), jnp.float32))

    t1 = _tc_t1(comp1, basis1.reshape(_B, _E))
    p1 = _sc_layer32(t1.reshape(_R * _N, 32), idx1c, dstc, scale,
                     jnp.zeros((640, 32), jnp.float32))
    x1, r2term = _tc_c1(p1, root1, bias1.reshape(1, 32), root2)

    t2 = _tc_t2(x1.reshape(625, 512), comp2, basis2)
    p2 = _sc_layer8(t2.reshape(_R * _N, 8), idx1c, dstc, scale,
                    jnp.zeros((640, 8), jnp.float32))
    x2, r3term = _tc_c2(p2, r2term, bias2.reshape(1, 8), root3)

    t3 = _tc_t3(comp3, basis3.reshape(_B, 8), x2)
    p3 = _sc_layer1(t3.reshape(_R * _N), idx1c, dstc, scale,
                    jnp.zeros((2000,), jnp.float32))
    return _tc_fin(p3.reshape(_NC, _N, 1), r3term, bias3.reshape(1, 1))


# speedup vs baseline: 116.1077x; 1.1034x over previous
import functools

import jax
import jax.numpy as jnp
from jax import lax
from jax.experimental import pallas as pl
from jax.experimental.pallas import tpu as pltpu
from jax.experimental.pallas import tpu_sc as plsc

_N = 10000
_R = 70
_B = 30
_E = 320000

_NC = 2
_NS = 16
_NW = _NC * _NS
_CH = 80
_EPT = _E // _NS
_EPW = _E // _NW
_CNT = _N * _R
_CNT_PAD = 700032
_ZCH = _CNT_PAD // _NS

_mesh = plsc.VectorSubcoreMesh(
    core_axis_name="c", subcore_axis_name="s", num_cores=_NC, num_subcores=_NS)
_sc_params = pltpu.CompilerParams(needs_layout_passes=False,
                                  use_tc_tiling_on_sc=False)


def _idx_body(s_ref, d_ref, t_ref, i1_ref, ic_ref):
    i1_ref[...] = t_ref[...] * _N + s_ref[...]
    ic_ref[...] = d_ref[...] * _R + t_ref[...]


_tc_idx = pl.pallas_call(
    _idx_body,
    out_shape=(jax.ShapeDtypeStruct((2500, 128), jnp.int32),
               jax.ShapeDtypeStruct((2500, 128), jnp.int32)))


def _t1_body(c_ref, b_ref, o_ref):
    o_ref[...] = jnp.dot(c_ref[...], b_ref[...],
                         preferred_element_type=jnp.float32,
                         precision=lax.Precision.HIGHEST)


_tc_t1 = pl.pallas_call(
    _t1_body,
    grid=(125,),
    in_specs=[pl.BlockSpec((_R, _B), lambda i: (0, 0)),
              pl.BlockSpec((_B, 2560), lambda i: (0, i))],
    out_specs=pl.BlockSpec((_R, 2560), lambda i: (0, i)),
    out_shape=jax.ShapeDtypeStruct((_R, _E), jnp.float32))


def _c1_body(p_ref, r1_ref, b1_ref, rt2_ref, x_ref, r2_ref):
    x = jnp.maximum(p_ref[0] + p_ref[1] + r1_ref[...] + b1_ref[...], 0.0)
    x_ref[...] = x
    r2_ref[...] = jnp.dot(x, rt2_ref[...], preferred_element_type=jnp.float32,
                         precision=lax.Precision.HIGHEST)


_tc_c1 = pl.pallas_call(
    _c1_body,
    out_shape=(jax.ShapeDtypeStruct((_N, 32), jnp.float32),
               jax.ShapeDtypeStruct((_N, 8), jnp.float32)))


def _t2_body(x_ref, c2_ref, b2_ref, o_ref):
    r = pl.program_id(0)
    rmask = lax.broadcasted_iota(jnp.int32, (_R, _B), 0) == r
    crow = jnp.sum(jnp.where(rmask, c2_ref[...], 0.0), axis=0)
    w2 = jnp.sum(crow[:, None, None] * b2_ref[...], axis=0)
    xs = x_ref[...]
    o_ref[...] = jnp.concatenate(
        [jnp.dot(xs[:, k * 32:(k + 1) * 32], w2,
                 preferred_element_type=jnp.float32,
                         precision=lax.Precision.HIGHEST) for k in range(16)],
        axis=1)[None]


_tc_t2 = pl.pallas_call(
    _t2_body,
    grid=(_R,),
    in_specs=[pl.BlockSpec((625, 512), lambda r: (0, 0)),
              pl.BlockSpec((_R, _B), lambda r: (0, 0)),
              pl.BlockSpec((_B, 32, 8), lambda r: (0, 0, 0))],
    out_specs=pl.BlockSpec((1, 625, 128), lambda r: (r, 0, 0)),
    out_shape=jax.ShapeDtypeStruct((_R, 625, 128), jnp.float32))


def _c2_body(p_ref, r2_ref, b2_ref, rt3_ref, x_ref, r3_ref):
    x = jnp.maximum(p_ref[0] + p_ref[1] + r2_ref[...] + b2_ref[...], 0.0)
    x_ref[...] = x
    r3_ref[...] = jnp.dot(x, rt3_ref[...], preferred_element_type=jnp.float32,
                         precision=lax.Precision.HIGHEST)


_tc_c2 = pl.pallas_call(
    _c2_body,
    out_shape=(jax.ShapeDtypeStruct((_N, 8), jnp.float32),
               jax.ShapeDtypeStruct((_N, 1), jnp.float32)))


def _t3_body(c3_ref, b3_ref, x_ref, o_ref):
    w3 = jnp.dot(c3_ref[...], b3_ref[...], preferred_element_type=jnp.float32,
                         precision=lax.Precision.HIGHEST)
    o_ref[...] = lax.dot_general(w3, x_ref[...], (((1,), (1,)), ((), ())),
                                 preferred_element_type=jnp.float32,
                         precision=lax.Precision.HIGHEST)


_tc_t3 = pl.pallas_call(
    _t3_body,
    out_shape=jax.ShapeDtypeStruct((_R, _N), jnp.float32))


def _fin_body(p_ref, r3_ref, b3_ref, o_ref):
    o_ref[...] = p_ref[0] + p_ref[1] + r3_ref[...] + b3_ref[...]


_tc_fin = pl.pallas_call(
    _fin_body,
    out_shape=jax.ShapeDtypeStruct((_N, 1), jnp.float32))


def _hist_body(idxc2_hbm, zeros_hbm, scale_hbm,
               ic2d, ic2g, ones_v, cvals, sall, zbuf, cnt_sp, sem):
    c = lax.axis_index("c")
    s = lax.axis_index("s")
    pltpu.sync_copy(zeros_hbm, zbuf)
    pltpu.sync_copy(zbuf, cnt_sp.at[pl.ds(s * _ZCH, _ZCH)])
    for g in range(_CH // 16):
        ones_v[pl.ds(g * 16, 16)] = jnp.ones((16,), jnp.float32)
    nch = _EPT // _CH
    ncw = _EPW // _CH
    w = c * _NS + s
    pltpu.sync_copy(idxc2_hbm.at[pl.ds(s * nch, nch)], ic2d)
    pltpu.sync_copy(idxc2_hbm.at[pl.ds(w * ncw, ncw)], ic2g)
    plsc.subcore_barrier()

    def hist_chunk(k, carry):
        pltpu.sync_copy(ones_v, cnt_sp.at[ic2d.at[k]], add=True)
        return carry

    lax.fori_loop(0, nch, hist_chunk, 0)
    plsc.subcore_barrier()

    def scale_chunk(k, carry):
        pltpu.async_copy(cnt_sp.at[ic2g.at[k]], cvals, sem).wait()
        for g in range(_CH // 16):
            v = cvals[pl.ds(g * 16, 16)]
            sall[pl.ds(k * _CH + g * 16, 16)] = 1.0 / jnp.maximum(v, 1.0)
        return carry

    lax.fori_loop(0, ncw, scale_chunk, 0)
    pltpu.sync_copy(sall, scale_hbm.at[pl.ds(w * _EPW, _EPW)])


_sc_hist = pl.kernel(
    _hist_body,
    out_type=jax.ShapeDtypeStruct((_E,), jnp.float32),
    mesh=_mesh,
    compiler_params=_sc_params,
    scratch_types=[
        pltpu.VMEM((_EPT // _CH, _CH), jnp.int32),
        pltpu.VMEM((_EPW // _CH, _CH), jnp.int32),
        pltpu.VMEM((_CH,), jnp.float32),
        pltpu.VMEM((_CH,), jnp.float32),
        pltpu.VMEM((_EPW,), jnp.float32),
        pltpu.VMEM((_ZCH,), jnp.float32),
        pltpu.VMEM_SHARED((_CNT_PAD,), jnp.float32),
        pltpu.SemaphoreType.DMA,
    ])


def _make_sc_layer(C):
    def body(t_hbm, idx12_hbm, dst2_hbm, scale_hbm, zeros_hbm,
             out_hbm, ig2d, d2d, sall, rows, rowsb, zobuf, acc_sp, sema, semb):
        c = lax.axis_index("c")
        s = lax.axis_index("s")
        w = c * _NS + s
        ncw = _EPW // _CH

        pltpu.sync_copy(zeros_hbm, zobuf)
        pltpu.sync_copy(idx12_hbm.at[pl.ds(w * ncw, ncw)], ig2d)
        pltpu.sync_copy(dst2_hbm.at[pl.ds(w * ncw, ncw)], d2d)
        pltpu.sync_copy(scale_hbm.at[pl.ds(w * _EPW, _EPW)], sall)

        @pl.when(s < 15)
        def _():
            pltpu.sync_copy(zobuf.at[pl.ds(0, 624)],
                            acc_sp.at[pl.ds(s * 624, 624)])

        @pl.when(s == 15)
        def _():
            pltpu.sync_copy(zobuf, acc_sp.at[pl.ds(9360, 640)])

        plsc.subcore_barrier()

        iota = lax.iota(jnp.int32, 16)

        def scale_scatter(k, rbuf):
            for p in range(_CH // 2):
                e = 2 * p + (iota >> 3)
                sc = plsc.load_gather(sall, [k * _CH + e])
                for cb in range(C // 8):
                    cv = cb * 8 + (iota & 7)
                    val = plsc.load_gather(rbuf, [e, cv])
                    plsc.store_scatter(rbuf, [e, cv], val * sc)
            pltpu.sync_copy(rbuf, acc_sp.at[d2d.at[k]], add=True)

        pltpu.async_copy(t_hbm.at[ig2d.at[0]], rows, sema)

        def pair(p, carry):
            ka = 2 * p
            kb = 2 * p + 1
            pltpu.async_copy(t_hbm.at[ig2d.at[kb]], rowsb, semb)
            pltpu.make_async_copy(t_hbm.at[ig2d.at[ka]], rows, sema).wait()
            scale_scatter(ka, rows)
            pltpu.async_copy(t_hbm.at[ig2d.at[ka + 2]], rows, sema)
            pltpu.make_async_copy(t_hbm.at[ig2d.at[kb]], rowsb, semb).wait()
            scale_scatter(kb, rowsb)
            return carry

        lax.fori_loop(0, (ncw - 1) // 2, pair, 0)
        pltpu.make_async_copy(t_hbm.at[ig2d.at[ncw - 1]], rows, sema).wait()
        scale_scatter(ncw - 1, rows)
        plsc.subcore_barrier()

        @pl.when(s < 15)
        def _():
            pltpu.sync_copy(acc_sp.at[pl.ds(s * 624, 624)],
                            zobuf.at[pl.ds(0, 624)])
            pltpu.sync_copy(zobuf.at[pl.ds(0, 624)],
                            out_hbm.at[c, pl.ds(s * 624, 624)])

        @pl.when(s == 15)
        def _():
            pltpu.sync_copy(acc_sp.at[pl.ds(9360, 640)], zobuf)
            pltpu.sync_copy(zobuf, out_hbm.at[c, pl.ds(9360, 640)])

    return pl.kernel(
        body,
        out_type=jax.ShapeDtypeStruct((_NC, _N, C), jnp.float32),
        mesh=_mesh,
        compiler_params=_sc_params,
        scratch_types=[
            pltpu.VMEM((_EPW // _CH, _CH), jnp.int32),
            pltpu.VMEM((_EPW // _CH, _CH), jnp.int32),
            pltpu.VMEM((_EPW,), jnp.float32),
            pltpu.VMEM((_CH, C), jnp.float32),
            pltpu.VMEM((_CH, C), jnp.float32),
            pltpu.VMEM((640, C), jnp.float32),
            pltpu.VMEM_SHARED((_N, C), jnp.float32),
            pltpu.SemaphoreType.DMA,
            pltpu.SemaphoreType.DMA,
        ])


_sc_layer32 = _make_sc_layer(32)
_sc_layer8 = _make_sc_layer(8)


def _l3_body(t_hbm, idx12_hbm, dst2_hbm, scale_hbm, zeros_hbm,
             out_hbm, ig2d, d2d, sall, rows, zobuf, acc_sp, sem):
    c = lax.axis_index("c")
    s = lax.axis_index("s")
    w = c * _NS + s
    ncw = _EPW // _CH
    pltpu.sync_copy(idx12_hbm.at[pl.ds(w * ncw, ncw)], ig2d)
    pltpu.sync_copy(dst2_hbm.at[pl.ds(w * ncw, ncw)], d2d)
    pltpu.sync_copy(scale_hbm.at[pl.ds(w * _EPW, _EPW)], sall)
    @pl.when(s < 5)
    def _():
        pltpu.sync_copy(zeros_hbm, zobuf)
        pltpu.sync_copy(zobuf, acc_sp.at[pl.ds(s * 2000, 2000)])
    plsc.subcore_barrier()

    def chunk(k, carry):
        pltpu.async_copy(t_hbm.at[ig2d.at[k]], rows, sem).wait()
        for g in range(_CH // 16):
            sl = pl.ds(g * 16, 16)
            ssl = pl.ds(k * _CH + g * 16, 16)
            rows[sl] = rows[sl] * sall[ssl]
        pltpu.sync_copy(rows, acc_sp.at[d2d.at[k]], add=True)
        return carry

    lax.fori_loop(0, ncw, chunk, 0)
    plsc.subcore_barrier()

    @pl.when(s < 5)
    def _():
        pltpu.sync_copy(acc_sp.at[pl.ds(s * 2000, 2000)], zobuf)
        pltpu.sync_copy(zobuf, out_hbm.at[pl.ds(c * _N + s * 2000, 2000)])


_sc_layer1 = pl.kernel(
    _l3_body,
    out_type=jax.ShapeDtypeStruct((_NC * _N,), jnp.float32),
    mesh=_mesh,
    compiler_params=_sc_params,
    scratch_types=[
        pltpu.VMEM((_EPW // _CH, _CH), jnp.int32),
        pltpu.VMEM((_EPW // _CH, _CH), jnp.int32),
        pltpu.VMEM((_EPW,), jnp.float32),
        pltpu.VMEM((_CH,), jnp.float32),
        pltpu.VMEM((2000,), jnp.float32),
        pltpu.VMEM_SHARED((_N,), jnp.float32),
        pltpu.SemaphoreType.DMA,
    ])


def kernel(edge_index, edge_type, basis1, comp1, root1, bias1,
           basis2, comp2, root2, bias2, basis3, comp3, root3, bias3):
    src = edge_index[0]
    dst = edge_index[1]

    idx1_2d, idxc_2d = _tc_idx(src.reshape(2500, 128),
                               dst.reshape(2500, 128),
                               edge_type.reshape(2500, 128))
    idx1c = idx1_2d.reshape(_E // _CH, _CH)
    idxcc = idxc_2d.reshape(_E // _CH, _CH)
    dstc = dst.reshape(_E // _CH, _CH)

    scale = _sc_hist(idxcc, jnp.zeros((_ZCH,), jnp.float32))

    t1 = _tc_t1(comp1, basis1.reshape(_B, _E))
    p1 = _sc_layer32(t1.reshape(_R * _N, 32), idx1c, dstc, scale,
                     jnp.zeros((640, 32), jnp.float32))
    x1, r2term = _tc_c1(p1, root1, bias1.reshape(1, 32), root2)

    t2 = _tc_t2(x1.reshape(625, 512), comp2, basis2)
    p2 = _sc_layer8(t2.reshape(_R * _N, 8), idx1c, dstc, scale,
                    jnp.zeros((640, 8), jnp.float32))
    x2, r3term = _tc_c2(p2, r2term, bias2.reshape(1, 8), root3)

    t3 = _tc_t3(comp3, basis3.reshape(_B, 8), x2)
    p3 = _sc_layer1(t3.reshape(_R * _N), idx1c, dstc, scale,
                    jnp.zeros((2000,), jnp.float32))
    return _tc_fin(p3.reshape(_NC, _N, 1), r3term, bias3.reshape(1, 1))
